# baseline (speedup 1.0000x reference)
import functools

import jax
import jax.numpy as jnp
from jax import lax
from jax.experimental import pallas as pl
from jax.experimental.pallas import tpu as pltpu
from jax.experimental.pallas import tpu_sc as plsc
from jax._src.pallas import mpmd


def _make_copy_kernel(rows: int, cols: int):
    n_cores = 2
    split = rows * 5 // 8

    scalar_mesh = plsc.ScalarSubcoreMesh(axis_name="c", num_cores=n_cores)
    vector_mesh = plsc.VectorSubcoreMesh(core_axis_name="c", subcore_axis_name="s")

    s_nbuf = 16
    s_rows_per_c = split // n_cores
    s_chunk = s_rows_per_c // s_nbuf

    n_workers = 32
    v_rows_per_w = (rows - split) // n_workers
    v_nbuf = 2
    v_chunk = v_rows_per_w // v_nbuf

    dma = pltpu.SemaphoreType.DMA.dtype

    scratch_types = [
        pltpu.VMEM_SHARED((s_nbuf, s_chunk, cols), jnp.float32),
        (pltpu.MemorySpace.SEMAPHORE @ scalar_mesh)((s_nbuf,), dma),
        (pltpu.MemorySpace.SEMAPHORE @ scalar_mesh)((), dma),
        (pltpu.MemorySpace.VMEM @ vector_mesh)((v_nbuf, v_chunk, cols), jnp.float32),
        (pltpu.MemorySpace.SEMAPHORE @ vector_mesh)((v_nbuf,), dma),
        (pltpu.MemorySpace.SEMAPHORE @ vector_mesh)((), dma),
    ]

    def scs_fn(table_hbm, out_hbm, sbuf, s_in_sems, s_out_sem, vbuf, v_in_sems, v_out_sem):
        del vbuf, v_in_sems, v_out_sem
        cid = lax.axis_index("c")
        base = cid * s_rows_per_c
        copies_in = []
        copies_out = []
        for b in range(s_nbuf):
            copies_in.append(
                pltpu.async_copy(
                    table_hbm.at[pl.ds(base + b * s_chunk, s_chunk)],
                    sbuf.at[b],
                    s_in_sems.at[b],
                )
            )
        for b in range(s_nbuf):
            copies_in[b].wait()
            copies_out.append(
                pltpu.async_copy(
                    sbuf.at[b],
                    out_hbm.at[pl.ds(base + b * s_chunk, s_chunk)],
                    s_out_sem,
                )
            )
        for b in range(s_nbuf):
            copies_out[b].wait()

    def tec_fn(table_hbm, out_hbm, sbuf, s_in_sems, s_out_sem, vbuf, v_in_sems, v_out_sem):
        del sbuf, s_in_sems, s_out_sem
        nc = lax.axis_size("c")
        wid = lax.axis_index("s") * nc + lax.axis_index("c")
        base = split + wid * v_rows_per_w
        copies_in = []
        copies_out = []
        for b in range(v_nbuf):
            copies_in.append(
                pltpu.async_copy(
                    table_hbm.at[pl.ds(base + b * v_chunk, v_chunk)],
                    vbuf.at[b],
                    v_in_sems.at[b],
                )
            )
        for b in range(v_nbuf):
            copies_in[b].wait()
            copies_out.append(
                pltpu.async_copy(
                    vbuf.at[b],
                    out_hbm.at[pl.ds(base + b * v_chunk, v_chunk)],
                    v_out_sem,
                )
            )
        for b in range(v_nbuf):
            copies_out[b].wait()

    return mpmd.mpmd_map(
        [(scalar_mesh, scs_fn), (vector_mesh, tec_fn)],
        out_types=jax.ShapeDtypeStruct((rows, cols), jnp.float32),
        scratch_types=scratch_types,
    )


def kernel(x, pos_table):
    seq_len = x.shape[-1]
    rows, cols = pos_table.shape
    assert seq_len == rows, "positions cover exactly the whole table"
    return _make_copy_kernel(rows, cols)(pos_table)


# speedup vs baseline: 1.0010x; 1.0010x over previous
import functools

import jax
import jax.numpy as jnp
from jax import lax
from jax.experimental import pallas as pl
from jax.experimental.pallas import tpu as pltpu
from jax.experimental.pallas import tpu_sc as plsc
from jax._src.pallas import mpmd


def _make_copy_kernel(rows: int, cols: int):
    n_cores = 2
    split = rows * 3 // 8

    scalar_mesh = plsc.ScalarSubcoreMesh(axis_name="c", num_cores=n_cores)
    vector_mesh = plsc.VectorSubcoreMesh(core_axis_name="c", subcore_axis_name="s")

    s_nbuf = 16
    s_rows_per_c = split // n_cores
    s_chunk = s_rows_per_c // s_nbuf

    n_workers = 32
    v_rows_per_w = (rows - split) // n_workers
    v_nbuf = 2
    v_chunk = v_rows_per_w // v_nbuf

    dma = pltpu.SemaphoreType.DMA.dtype

    scratch_types = [
        pltpu.VMEM_SHARED((s_nbuf, s_chunk, cols), jnp.float32),
        (pltpu.MemorySpace.SEMAPHORE @ scalar_mesh)((s_nbuf,), dma),
        (pltpu.MemorySpace.SEMAPHORE @ scalar_mesh)((), dma),
        (pltpu.MemorySpace.VMEM @ vector_mesh)((v_nbuf, v_chunk, cols), jnp.float32),
        (pltpu.MemorySpace.SEMAPHORE @ vector_mesh)((v_nbuf,), dma),
        (pltpu.MemorySpace.SEMAPHORE @ vector_mesh)((), dma),
    ]

    def scs_fn(table_hbm, out_hbm, sbuf, s_in_sems, s_out_sem, vbuf, v_in_sems, v_out_sem):
        del vbuf, v_in_sems, v_out_sem
        cid = lax.axis_index("c")
        base = cid * s_rows_per_c
        copies_in = []
        copies_out = []
        for b in range(s_nbuf):
            copies_in.append(
                pltpu.async_copy(
                    table_hbm.at[pl.ds(base + b * s_chunk, s_chunk)],
                    sbuf.at[b],
                    s_in_sems.at[b],
                )
            )
        for b in range(s_nbuf):
            copies_in[b].wait()
            copies_out.append(
                pltpu.async_copy(
                    sbuf.at[b],
                    out_hbm.at[pl.ds(base + b * s_chunk, s_chunk)],
                    s_out_sem,
                )
            )
        for b in range(s_nbuf):
            copies_out[b].wait()

    def tec_fn(table_hbm, out_hbm, sbuf, s_in_sems, s_out_sem, vbuf, v_in_sems, v_out_sem):
        del sbuf, s_in_sems, s_out_sem
        nc = lax.axis_size("c")
        wid = lax.axis_index("s") * nc + lax.axis_index("c")
        base = split + wid * v_rows_per_w
        copies_in = []
        copies_out = []
        for b in range(v_nbuf):
            copies_in.append(
                pltpu.async_copy(
                    table_hbm.at[pl.ds(base + b * v_chunk, v_chunk)],
                    vbuf.at[b],
                    v_in_sems.at[b],
                )
            )
        for b in range(v_nbuf):
            copies_in[b].wait()
            copies_out.append(
                pltpu.async_copy(
                    vbuf.at[b],
                    out_hbm.at[pl.ds(base + b * v_chunk, v_chunk)],
                    v_out_sem,
                )
            )
        for b in range(v_nbuf):
            copies_out[b].wait()

    return mpmd.mpmd_map(
        [(scalar_mesh, scs_fn), (vector_mesh, tec_fn)],
        out_types=jax.ShapeDtypeStruct((rows, cols), jnp.float32),
        scratch_types=scratch_types,
    )


def kernel(x, pos_table):
    seq_len = x.shape[-1]
    rows, cols = pos_table.shape
    assert seq_len == rows, "positions cover exactly the whole table"
    return _make_copy_kernel(rows, cols)(pos_table)


# speedup vs baseline: 1.0137x; 1.0126x over previous
import functools

import jax
import jax.numpy as jnp
from jax import lax
from jax.experimental import pallas as pl
from jax.experimental.pallas import tpu as pltpu
from jax.experimental.pallas import tpu_sc as plsc
from jax._src.pallas import mpmd


def _make_copy_kernel(rows: int, cols: int):
    n_cores = 2
    half = rows // 2

    scalar_mesh = plsc.ScalarSubcoreMesh(axis_name="c", num_cores=n_cores)
    vector_mesh = plsc.VectorSubcoreMesh(core_axis_name="c", subcore_axis_name="s")

    s_nbuf = 16
    s_rows_per_c = half // n_cores
    s_chunk = s_rows_per_c // s_nbuf

    n_workers = 32
    v_rows_per_w = half // n_workers
    v_nbuf = 2
    v_chunk = v_rows_per_w // v_nbuf

    dma = pltpu.SemaphoreType.DMA.dtype

    scratch_types = [
        pltpu.VMEM_SHARED((s_nbuf, s_chunk, cols), jnp.float32),
        (pltpu.MemorySpace.SEMAPHORE @ scalar_mesh)((s_nbuf,), dma),
        (pltpu.MemorySpace.SEMAPHORE @ scalar_mesh)((), dma),
        (pltpu.MemorySpace.VMEM @ vector_mesh)((v_nbuf, v_chunk, cols), jnp.float32),
        (pltpu.MemorySpace.SEMAPHORE @ vector_mesh)((v_nbuf,), dma),
        (pltpu.MemorySpace.SEMAPHORE @ vector_mesh)((), dma),
    ]

    def scs_fn(table_hbm, out_hbm, sbuf, s_in_sems, s_out_sem, vbuf, v_in_sems, v_out_sem):
        del vbuf, v_in_sems, v_out_sem
        cid = lax.axis_index("c")
        base = cid * s_rows_per_c
        copies_in = []
        copies_out = []
        for b in range(s_nbuf):
            copies_in.append(
                pltpu.async_copy(
                    table_hbm.at[pl.ds(base + b * s_chunk, s_chunk)],
                    sbuf.at[b],
                    s_in_sems.at[b],
                )
            )
        for b in range(s_nbuf):
            copies_in[b].wait()
            copies_out.append(
                pltpu.async_copy(
                    sbuf.at[b],
                    out_hbm.at[pl.ds(base + b * s_chunk, s_chunk)],
                    s_out_sem,
                )
            )
        for b in range(s_nbuf):
            copies_out[b].wait()

    def tec_fn(table_hbm, out_hbm, sbuf, s_in_sems, s_out_sem, vbuf, v_in_sems, v_out_sem):
        del sbuf, s_in_sems, s_out_sem
        nc = lax.axis_size("c")
        wid = lax.axis_index("s") * nc + lax.axis_index("c")
        base = half + wid * v_rows_per_w
        copies_in = []
        copies_out = []
        for b in range(v_nbuf):
            copies_in.append(
                pltpu.async_copy(
                    table_hbm.at[pl.ds(base + b * v_chunk, v_chunk)],
                    vbuf.at[b],
                    v_in_sems.at[b],
                )
            )
        for b in range(v_nbuf):
            copies_in[b].wait()
            copies_out.append(
                pltpu.async_copy(
                    vbuf.at[b],
                    out_hbm.at[pl.ds(base + b * v_chunk, v_chunk)],
                    v_out_sem,
                )
            )
        for b in range(v_nbuf):
            copies_out[b].wait()

    return mpmd.mpmd_map(
        [(scalar_mesh, scs_fn), (vector_mesh, tec_fn)],
        out_types=jax.ShapeDtypeStruct((rows, cols), jnp.float32),
        scratch_types=scratch_types,
    )


def kernel(x, pos_table):
    seq_len = x.shape[-1]
    rows, cols = pos_table.shape
    assert seq_len == rows, "positions cover exactly the whole table"
    return _make_copy_kernel(rows, cols)(pos_table)
